# Initial kernel scaffold; baseline (speedup 1.0000x reference)
import jax
import jax.numpy as jnp
from jax.experimental import pallas as pl


def kernel(pred_joints2d, gt_joints3d, proj_matrix, valid):
    raise NotImplementedError("write your pallas kernel here")


# speedup vs baseline: 2678.7490x; 2678.7490x over previous
import jax
import jax.numpy as jnp
from jax.experimental import pallas as pl
from jax.experimental.pallas import tpu as pltpu

_IMAGE_W, _IMAGE_H = 384.0, 512.0
_BETA = 0.05
_NVIEW = 8
_NSWEEPS = 5
_SKIP_TOL2 = 1e-14
_PAIRS = [(0, 1), (0, 2), (0, 3), (1, 2), (1, 3), (2, 3)]


def _two_sum(a, b):
    s = a + b
    b2 = s - a
    a2 = s - b2
    return s, (a - a2) + (b - b2)


def _split(a):
    c = 4097.0 * a
    hi = c - (c - a)
    return hi, a - hi


def _two_prod(a, b):
    p = a * b
    ahi, alo = _split(a)
    bhi, blo = _split(b)
    e = ((ahi * bhi - p) + ahi * blo + alo * bhi) + alo * blo
    return p, e


def _ds_acc(hi, lo, p, e):
    s, err = _two_sum(hi, p)
    return s, lo + (err + e)


def _dlt_loss_kernel(pred_ref, proj_ref, valid_ref, gt_ref, out_ref):
    bb = valid_ref.shape[-1]

    def a_rows(v):
        x = ((pred_ref[0, v] + 1.0) * 0.5) * _IMAGE_W
        y = ((pred_ref[1, v] + 1.0) * 0.5) * _IMAGE_H
        w = valid_ref[v]
        r0, r1 = [], []
        for c in range(4):
            p0 = proj_ref[v, 0, c:c + 1, :]
            p1 = proj_ref[v, 1, c:c + 1, :]
            p2 = proj_ref[v, 2, c:c + 1, :]
            r0.append((x * p2 - p0) * w)
            r1.append((y * p2 - p1) * w)
        return r0, r1

    cols = [[None] * (2 * _NVIEW) for _ in range(4)]
    for v in range(_NVIEW):
        r0, r1 = a_rows(v)
        for c in range(4):
            cols[c][2 * v] = r0[c]
            cols[c][2 * v + 1] = r1[c]

    nrm = [None] * 4
    for c in range(4):
        acc = cols[c][0] * cols[c][0]
        for r in range(1, 2 * _NVIEW):
            acc = acc + cols[c][r] * cols[c][r]
        nrm[c] = acc

    vm = [[1.0 if i == c else 0.0 for c in range(4)] for i in range(4)]

    for _ in range(_NSWEEPS):
        for (p, q) in _PAIRS:
            gam = cols[p][0] * cols[q][0]
            for r in range(1, 2 * _NVIEW):
                gam = gam + cols[p][r] * cols[q][r]
            app = nrm[p]
            aqq = nrm[q]
            skip = gam * gam <= _SKIP_TOL2 * (app * aqq)
            denom = 2.0 * gam
            tau = (aqq - app) / jnp.where(denom == 0.0, 1.0, denom)
            sgn = jnp.where(tau >= 0.0, 1.0, -1.0)
            t = sgn / (jnp.abs(tau) + jnp.sqrt(1.0 + tau * tau))
            t = jnp.where(skip, 0.0, t)
            c_ = jax.lax.rsqrt(1.0 + t * t)
            s_ = t * c_
            for r in range(2 * _NVIEW):
                ap = cols[p][r]
                aq = cols[q][r]
                cols[p][r] = c_ * ap - s_ * aq
                cols[q][r] = s_ * ap + c_ * aq
            nrm[p] = app - t * gam
            nrm[q] = aqq + t * gam
            for i in range(4):
                vp = vm[i][p]
                vq = vm[i][q]
                vm[i][p] = c_ * vp - s_ * vq
                vm[i][q] = s_ * vp + c_ * vq

    lam = nrm[0]
    xs = [vm[a][0] for a in range(4)]
    for k in range(1, 4):
        cond = nrm[k] < lam
        for a in range(4):
            xs[a] = jnp.where(cond, vm[a][k], xs[a])
        lam = jnp.minimum(lam, nrm[k])

    zeros = jnp.zeros((valid_ref.shape[1], bb), jnp.float32)
    mpairs = [(a, b) for a in range(4) for b in range(a, 4)]
    mhi = {ab: zeros for ab in mpairs}
    mlo = {ab: zeros for ab in mpairs}
    for v in range(_NVIEW):
        for row in a_rows(v):
            for (a, b) in mpairs:
                p_, e_ = _two_prod(row[a], row[b])
                mhi[(a, b)], mlo[(a, b)] = _ds_acc(mhi[(a, b)], mlo[(a, b)], p_, e_)

    def m_ds(a, b):
        ab = (a, b) if a <= b else (b, a)
        return mhi[ab], mlo[ab]

    hhi, hlo = [], []
    for a in range(4):
        hi = zeros
        lo = zeros
        for b in range(4):
            m_h, m_l = m_ds(a, b)
            p_, e_ = _two_prod(m_h, xs[b])
            e_ = e_ + m_l * xs[b]
            hi, lo = _ds_acc(hi, lo, p_, e_)
        hhi.append(hi)
        hlo.append(lo)

    xn = list(xs)
    for k in range(4):
        hi = zeros
        lo = zeros
        for a in range(4):
            p_, e_ = _two_prod(hhi[a], vm[a][k])
            e_ = e_ + hlo[a] * vm[a][k]
            hi, lo = _ds_acc(hi, lo, p_, e_)
        ok = nrm[k] > lam
        t = jnp.where(ok, -hi / jnp.where(ok, nrm[k], 1.0), 0.0)
        t = jnp.where(jnp.abs(t) < 1e-2, t, 0.0)
        for a in range(4):
            xn[a] = xn[a] + t * vm[a][k]

    inv_w = 1.0 / xn[3]
    total = None
    for a in range(3):
        pred = xn[a] * inv_w
        d = jnp.abs(pred - gt_ref[a])
        l = jnp.where(d < _BETA, (0.5 / _BETA) * d * d, d - 0.5 * _BETA)
        total = l if total is None else total + l
    part = jnp.sum(total, axis=0, keepdims=True)
    out_ref[...] = part.reshape(1, 1, bb)


def kernel(pred_joints2d, gt_joints3d, proj_matrix, valid):
    b, v, j, _ = pred_joints2d.shape
    bb = min(1024, b)
    nb = b // bb

    pred_t = jnp.transpose(pred_joints2d, (3, 1, 2, 0))
    proj_t = jnp.transpose(proj_matrix, (1, 2, 3, 0))
    valid_t = jnp.transpose(valid, (1, 2, 0))
    gt_t = jnp.transpose(gt_joints3d, (2, 1, 0))

    out = pl.pallas_call(
        _dlt_loss_kernel,
        grid=(nb,),
        in_specs=[
            pl.BlockSpec((2, v, j, bb), lambda i: (0, 0, 0, i)),
            pl.BlockSpec((v, 3, 4, bb), lambda i: (0, 0, 0, i)),
            pl.BlockSpec((v, j, bb), lambda i: (0, 0, i)),
            pl.BlockSpec((3, j, bb), lambda i: (0, 0, i)),
        ],
        out_specs=pl.BlockSpec((1, 1, bb), lambda i: (i, 0, 0)),
        out_shape=jax.ShapeDtypeStruct((nb, 1, bb), jnp.float32),
        compiler_params=pltpu.CompilerParams(
            dimension_semantics=("parallel",),
            vmem_limit_bytes=48 * 1024 * 1024,
        ),
    )(pred_t, proj_t, valid_t, gt_t)

    return out.sum() / b
